# baseline (speedup 1.0000x reference)
import functools

import jax
import jax.numpy as jnp
from jax import lax
from jax.experimental import pallas as pl
from jax.experimental.pallas import tpu as pltpu
from jax.experimental.pallas import tpu_sc as plsc

D_MODEL = 64
SCALE = 8.0

_LANES = 16
_CH = 128


@functools.lru_cache(maxsize=None)
def _build(n_ch_total: int, V: int, D: int):
    info = plsc.get_sparse_core_info()
    NC, NS = info.num_cores, info.num_subcores
    NW = NC * NS
    n_ch_w = n_ch_total // NW
    n_pairs = n_ch_w // 2
    B = n_ch_total * _CH
    mesh = plsc.VectorSubcoreMesh(core_axis_name="c", subcore_axis_name="s")

    @functools.partial(
        pl.kernel,
        out_type=jax.ShapeDtypeStruct((B, D), jnp.float32),
        mesh=mesh,
        compiler_params=pltpu.CompilerParams(use_tc_tiling_on_sc=False),
        scratch_types=[
            pltpu.VMEM((n_ch_w, _CH), jnp.int32),
            pltpu.VMEM((_CH, D), jnp.float32),
            pltpu.VMEM((_CH, D), jnp.float32),
            pltpu.SemaphoreType.DMA,
            pltpu.SemaphoreType.DMA,
        ],
    )
    def emb(idx_hbm, table_hbm, out_hbm, idx_v, buf0, buf1, sem0, sem1):
        wid = lax.axis_index("s") * NC + lax.axis_index("c")
        row0 = wid * n_ch_w
        pltpu.sync_copy(idx_hbm.at[pl.ds(row0, n_ch_w)], idx_v)

        bufs = (buf0, buf1)
        sems = (sem0, sem1)

        def start_gather(c, b):
            pltpu.async_copy(table_hbm.at[idx_v.at[c]], bufs[b], sems[b])

        def finish_chunk(c, b):
            pltpu.make_async_copy(
                table_hbm.at[idx_v.at[c]], bufs[b], sems[b]).wait()
            buf = bufs[b]

            def scale_row(r, _):
                for j in range(D // _LANES):
                    sl = pl.ds(j * _LANES, _LANES)
                    buf[r, sl] = buf[r, sl] * SCALE
                return _

            lax.fori_loop(0, _CH, scale_row, 0, unroll=2)
            pltpu.sync_copy(buf, out_hbm.at[pl.ds((row0 + c) * _CH, _CH)])

        start_gather(0, 0)
        start_gather(1, 1)

        def pair(g, _):
            for b in range(2):
                c = 2 * g + b
                finish_chunk(c, b)
                start_gather(c + 2, b)
            return _

        lax.fori_loop(0, n_pairs - 1, pair, 0)
        finish_chunk(n_ch_w - 2, 0)
        finish_chunk(n_ch_w - 1, 1)

    return emb


def kernel(x, table):
    B0, B1 = x.shape
    V, D = table.shape
    B = B0 * B1
    n_ch_total = B // _CH
    xf = x.reshape(n_ch_total, _CH).astype(jnp.int32)
    out = _build(n_ch_total, V, D)(xf, table)
    return out.reshape(B0, B1, D)
